# baseline (speedup 1.0000x reference)
import functools

import jax
import jax.numpy as jnp
from jax import lax
from jax.experimental import pallas as pl
from jax.experimental.pallas import tpu as pltpu
from jax.experimental.pallas import tpu_sc as plsc

BATCH = 4
SEQ = 4096
VOCAB = 100000
K = 2
TOKENS = BATCH * SEQ
NUM_WORKERS = 32
TPW = TOKENS // NUM_WORKERS
ECH = 128
TCH = ECH // K
NCH = TPW // TCH
LANES = 16

_mesh = plsc.VectorSubcoreMesh(core_axis_name="c", subcore_axis_name="s")


@functools.partial(
    pl.kernel,
    mesh=_mesh,
    compiler_params=pltpu.CompilerParams(
        use_tc_tiling_on_sc=False, needs_layout_passes=False
    ),
    out_type=jax.ShapeDtypeStruct((TOKENS * K,), jnp.int32),
    scratch_types=[
        pltpu.VMEM((TPW,), jnp.int32),
        pltpu.VMEM((NCH, ECH), jnp.int32),
        pltpu.VMEM((NCH, ECH), jnp.int32),
        pltpu.SemaphoreType.DMA,
    ],
)
def _hash_gather(idx_hbm, table_hbm, out_hbm, idx_v, eidx_v, rows_v, sem):
    wid = lax.axis_index("s") * 2 + lax.axis_index("c")
    base_t = wid * TPW
    base_o = wid * TPW * K
    pltpu.sync_copy(idx_hbm.at[pl.ds(base_t, TPW)], idx_v)
    lane = lax.iota(jnp.int32, LANES)
    lane2 = lane + lane
    one = jnp.int32(1)
    for j in range(NCH):
        row = jnp.full((LANES,), j, jnp.int32)
        for g in range(TCH // LANES):
            a = idx_v[pl.ds(j * TCH + g * LANES, LANES)]
            a2 = a + a
            col = lane2 + jnp.int32(2 * g * LANES)
            plsc.store_scatter(eidx_v, [row, col], a2)
            plsc.store_scatter(eidx_v, [row, col + one], a2 + one)
    copies = [
        pltpu.make_async_copy(
            table_hbm.at[eidx_v.at[jnp.int32(j)]], rows_v.at[jnp.int32(j)], sem
        )
        for j in range(NCH)
    ]
    for c in copies:
        c.start()
    for c in copies:
        c.wait()
    for j in range(NCH):
        pltpu.sync_copy(
            rows_v.at[jnp.int32(j)], out_hbm.at[pl.ds(base_o + j * ECH, ECH)]
        )


def kernel(input, hash):
    idx = input.astype(jnp.int32).reshape(TOKENS)
    table = hash.astype(jnp.int32).reshape(VOCAB * K)
    out = _hash_gather(idx, table)
    return out.reshape(BATCH, SEQ, K).astype(hash.dtype)


# speedup vs baseline: 8.6593x; 8.6593x over previous
import functools

import jax
import jax.numpy as jnp
from jax import lax
from jax.experimental import pallas as pl
from jax.experimental.pallas import tpu as pltpu
from jax.experimental.pallas import tpu_sc as plsc

BATCH = 4
SEQ = 4096
VOCAB = 100000
K = 2
TOKENS = BATCH * SEQ
NUM_WORKERS = 32
TPW = TOKENS // NUM_WORKERS
CHUNK = 128
NCH = TPW // CHUNK

_mesh = plsc.VectorSubcoreMesh(core_axis_name="c", subcore_axis_name="s")


@functools.partial(
    pl.kernel,
    mesh=_mesh,
    compiler_params=pltpu.CompilerParams(
        use_tc_tiling_on_sc=False, needs_layout_passes=False
    ),
    out_type=(
        jax.ShapeDtypeStruct((TOKENS // CHUNK, CHUNK), jnp.int32),
        jax.ShapeDtypeStruct((TOKENS // CHUNK, CHUNK), jnp.int32),
    ),
    scratch_types=[
        pltpu.VMEM((NCH, CHUNK), jnp.int32),
        pltpu.VMEM((NCH, CHUNK), jnp.int32),
        pltpu.VMEM((NCH, CHUNK), jnp.int32),
        pltpu.SemaphoreType.DMA,
    ],
)
def _hash_gather(idx_hbm, table_hbm, out0_hbm, out1_hbm, idx_v, r0_v, r1_v, sem):
    wid = lax.axis_index("s") * 2 + lax.axis_index("c")
    pltpu.sync_copy(idx_hbm.at[pl.ds(wid * NCH, NCH)], idx_v)
    tab1 = table_hbm.at[pl.ds(VOCAB, VOCAB)]
    copies = []
    for j in range(NCH):
        j32 = jnp.int32(j)
        copies.append(
            pltpu.make_async_copy(table_hbm.at[idx_v.at[j32]], r0_v.at[j32], sem)
        )
        copies.append(
            pltpu.make_async_copy(tab1.at[idx_v.at[j32]], r1_v.at[j32], sem)
        )
    for c in copies:
        c.start()
    for c in copies:
        c.wait()
    pltpu.sync_copy(r0_v, out0_hbm.at[pl.ds(wid * NCH, NCH)])
    pltpu.sync_copy(r1_v, out1_hbm.at[pl.ds(wid * NCH, NCH)])


def kernel(input, hash):
    idx = input.astype(jnp.int32).reshape(TOKENS // CHUNK, CHUNK)
    table = jnp.concatenate([hash[:, 0], hash[:, 1]]).astype(jnp.int32)
    r0, r1 = _hash_gather(idx, table)
    h0 = r0.astype(hash.dtype).reshape(BATCH, SEQ)
    h1 = r1.astype(hash.dtype).reshape(BATCH, SEQ)
    return jnp.stack([h0, h1], axis=-1)


# speedup vs baseline: 8.8979x; 1.0276x over previous
import functools

import jax
import jax.numpy as jnp
from jax import lax
from jax.experimental import pallas as pl
from jax.experimental.pallas import tpu as pltpu
from jax.experimental.pallas import tpu_sc as plsc

BATCH = 4
SEQ = 4096
VOCAB = 100000
K = 2
TOKENS = BATCH * SEQ
NUM_WORKERS = 32
TPW = TOKENS // NUM_WORKERS
CHUNK = 128
NCH = TPW // CHUNK

_mesh = plsc.VectorSubcoreMesh(core_axis_name="c", subcore_axis_name="s")


@functools.partial(
    pl.kernel,
    mesh=_mesh,
    compiler_params=pltpu.CompilerParams(
        use_tc_tiling_on_sc=False, needs_layout_passes=False
    ),
    out_type=(
        jax.ShapeDtypeStruct((TOKENS // CHUNK, CHUNK), jnp.int32),
        jax.ShapeDtypeStruct((TOKENS // CHUNK, CHUNK), jnp.int32),
    ),
    scratch_types=[
        pltpu.VMEM((NCH, CHUNK), jnp.int32),
        pltpu.VMEM((NCH, CHUNK), jnp.int32),
        pltpu.VMEM((NCH, CHUNK), jnp.int32),
        pltpu.SemaphoreType.DMA,
    ],
)
def _hash_gather(idx_hbm, tab0_hbm, tab1_hbm, out0_hbm, out1_hbm, idx_v, r0_v, r1_v, sem):
    wid = lax.axis_index("s") * 2 + lax.axis_index("c")
    pltpu.sync_copy(idx_hbm.at[pl.ds(wid * NCH, NCH)], idx_v)
    copies = []
    for j in range(NCH):
        j32 = jnp.int32(j)
        copies.append(
            pltpu.make_async_copy(tab0_hbm.at[idx_v.at[j32]], r0_v.at[j32], sem)
        )
        copies.append(
            pltpu.make_async_copy(tab1_hbm.at[idx_v.at[j32]], r1_v.at[j32], sem)
        )
    for c in copies:
        c.start()
    for c in copies:
        c.wait()
    pltpu.sync_copy(r0_v, out0_hbm.at[pl.ds(wid * NCH, NCH)])
    pltpu.sync_copy(r1_v, out1_hbm.at[pl.ds(wid * NCH, NCH)])


def kernel(input, hash):
    idx = input.astype(jnp.int32).reshape(TOKENS // CHUNK, CHUNK)
    tab0 = hash[:, 0].astype(jnp.int32)
    tab1 = hash[:, 1].astype(jnp.int32)
    r0, r1 = _hash_gather(idx, tab0, tab1)
    h0 = r0.astype(jnp.uint32).astype(hash.dtype).reshape(BATCH, SEQ)
    h1 = r1.astype(jnp.uint32).astype(hash.dtype).reshape(BATCH, SEQ)
    return jnp.stack([h0, h1], axis=-1)


# speedup vs baseline: 8.9447x; 1.0053x over previous
import functools

import jax
import jax.numpy as jnp
from jax import lax
from jax.experimental import pallas as pl
from jax.experimental.pallas import tpu as pltpu
from jax.experimental.pallas import tpu_sc as plsc

BATCH = 4
SEQ = 4096
VOCAB = 100000
K = 2
TOKENS = BATCH * SEQ
NUM_WORKERS = 32
TPW = TOKENS // NUM_WORKERS
CHUNK = 128
NCH = TPW // CHUNK

_mesh = plsc.VectorSubcoreMesh(core_axis_name="c", subcore_axis_name="s")


@functools.partial(
    pl.kernel,
    mesh=_mesh,
    compiler_params=pltpu.CompilerParams(
        use_tc_tiling_on_sc=False, needs_layout_passes=False
    ),
    out_type=(
        jax.ShapeDtypeStruct((TOKENS,), jnp.int32),
        jax.ShapeDtypeStruct((TOKENS,), jnp.int32),
    ),
    scratch_types=[
        pltpu.VMEM((TPW,), jnp.int32),
        pltpu.VMEM((TPW,), jnp.int32),
        pltpu.VMEM((TPW,), jnp.int32),
        pltpu.SemaphoreType.DMA,
    ],
)
def _hash_gather(idx_hbm, tab0_hbm, tab1_hbm, out0_hbm, out1_hbm, idx_v, r0_v, r1_v, sem):
    wid = lax.axis_index("s") * 2 + lax.axis_index("c")
    pltpu.sync_copy(idx_hbm.at[pl.ds(wid * TPW, TPW)], idx_v)
    copies = [
        pltpu.make_async_copy(tab0_hbm.at[idx_v], r0_v, sem),
        pltpu.make_async_copy(tab1_hbm.at[idx_v], r1_v, sem),
    ]
    for c in copies:
        c.start()
    for c in copies:
        c.wait()
    pltpu.sync_copy(r0_v, out0_hbm.at[pl.ds(wid * TPW, TPW)])
    pltpu.sync_copy(r1_v, out1_hbm.at[pl.ds(wid * TPW, TPW)])


def kernel(input, hash):
    idx = input.astype(jnp.int32).reshape(TOKENS)
    tab0 = hash[:, 0].astype(jnp.int32)
    tab1 = hash[:, 1].astype(jnp.int32)
    r0, r1 = _hash_gather(idx, tab0, tab1)
    h0 = r0.astype(jnp.uint32).astype(hash.dtype).reshape(BATCH, SEQ)
    h1 = r1.astype(jnp.uint32).astype(hash.dtype).reshape(BATCH, SEQ)
    return jnp.stack([h0, h1], axis=-1)


# speedup vs baseline: 8.9753x; 1.0034x over previous
import functools

import jax
import jax.numpy as jnp
from jax import lax
from jax.experimental import pallas as pl
from jax.experimental.pallas import tpu as pltpu
from jax.experimental.pallas import tpu_sc as plsc

BATCH = 4
SEQ = 4096
VOCAB = 100000
K = 2
TOKENS = BATCH * SEQ
NUM_WORKERS = 32
TPW = TOKENS // NUM_WORKERS
CHUNK = 128
NCH = TPW // CHUNK

_mesh = plsc.VectorSubcoreMesh(core_axis_name="c", subcore_axis_name="s")


@functools.partial(
    pl.kernel,
    mesh=_mesh,
    compiler_params=pltpu.CompilerParams(
        use_tc_tiling_on_sc=False, needs_layout_passes=False
    ),
    out_type=(
        jax.ShapeDtypeStruct((TOKENS,), jnp.int32),
        jax.ShapeDtypeStruct((TOKENS,), jnp.int32),
    ),
    scratch_types=[
        pltpu.VMEM((TPW,), jnp.int32),
        pltpu.VMEM((TPW,), jnp.int32),
        pltpu.VMEM((TPW,), jnp.int32),
        pltpu.SemaphoreType.DMA,
        pltpu.SemaphoreType.DMA,
        pltpu.SemaphoreType.DMA,
    ],
)
def _hash_gather(
    idx_hbm, tab0_hbm, tab1_hbm, out0_hbm, out1_hbm,
    idx_v, r0_v, r1_v, sem_i, sem0, sem1,
):
    wid = lax.axis_index("s") * 2 + lax.axis_index("c")
    base = wid * TPW
    idx_cp = pltpu.make_async_copy(idx_hbm.at[pl.ds(base, TPW)], idx_v, sem_i)
    idx_cp.start()
    g0 = pltpu.make_async_copy(tab0_hbm.at[idx_v], r0_v, sem0)
    g1 = pltpu.make_async_copy(tab1_hbm.at[idx_v], r1_v, sem1)
    idx_cp.wait()
    g0.start()
    g1.start()
    g0.wait()
    o0 = pltpu.make_async_copy(r0_v, out0_hbm.at[pl.ds(base, TPW)], sem0)
    o0.start()
    g1.wait()
    o1 = pltpu.make_async_copy(r1_v, out1_hbm.at[pl.ds(base, TPW)], sem1)
    o1.start()
    o0.wait()
    o1.wait()


def kernel(input, hash):
    idx = input.astype(jnp.int32).reshape(TOKENS)
    tab0 = hash[:, 0].astype(jnp.int32)
    tab1 = hash[:, 1].astype(jnp.int32)
    r0, r1 = _hash_gather(idx, tab0, tab1)
    h0 = r0.astype(jnp.uint32).astype(hash.dtype).reshape(BATCH, SEQ)
    h1 = r1.astype(jnp.uint32).astype(hash.dtype).reshape(BATCH, SEQ)
    return jnp.stack([h0, h1], axis=-1)


# speedup vs baseline: 9.2318x; 1.0286x over previous
import functools

import jax
import jax.numpy as jnp
from jax import lax
from jax.experimental import pallas as pl
from jax.experimental.pallas import tpu as pltpu
from jax.experimental.pallas import tpu_sc as plsc

BATCH = 4
SEQ = 4096
VOCAB = 100000
K = 2
TOKENS = BATCH * SEQ
NUM_WORKERS = 32
TPW = TOKENS // NUM_WORKERS
CHUNK = 128
NCH = TPW // CHUNK

_mesh = plsc.VectorSubcoreMesh(core_axis_name="c", subcore_axis_name="s")


@functools.partial(
    pl.kernel,
    mesh=_mesh,
    compiler_params=pltpu.CompilerParams(
        use_tc_tiling_on_sc=False, needs_layout_passes=False
    ),
    out_type=(
        jax.ShapeDtypeStruct((TOKENS,), jnp.uint32),
        jax.ShapeDtypeStruct((TOKENS,), jnp.uint32),
    ),
    scratch_types=[
        pltpu.VMEM((TPW,), jnp.int32),
        pltpu.VMEM((TPW,), jnp.uint32),
        pltpu.VMEM((TPW,), jnp.uint32),
        pltpu.SemaphoreType.DMA,
        pltpu.SemaphoreType.DMA,
        pltpu.SemaphoreType.DMA,
    ],
)
def _hash_gather(
    idx_hbm, tab0_hbm, tab1_hbm, out0_hbm, out1_hbm,
    idx_v, r0_v, r1_v, sem_i, sem0, sem1,
):
    wid = lax.axis_index("s") * 2 + lax.axis_index("c")
    base = wid * TPW
    idx_cp = pltpu.make_async_copy(idx_hbm.at[pl.ds(base, TPW)], idx_v, sem_i)
    idx_cp.start()
    g0 = pltpu.make_async_copy(tab0_hbm.at[idx_v], r0_v, sem0)
    g1 = pltpu.make_async_copy(tab1_hbm.at[idx_v], r1_v, sem1)
    idx_cp.wait()
    g0.start()
    g1.start()
    g0.wait()
    o0 = pltpu.make_async_copy(r0_v, out0_hbm.at[pl.ds(base, TPW)], sem0)
    o0.start()
    g1.wait()
    o1 = pltpu.make_async_copy(r1_v, out1_hbm.at[pl.ds(base, TPW)], sem1)
    o1.start()
    o0.wait()
    o1.wait()


def kernel(input, hash):
    idx = input.astype(jnp.int32).reshape(TOKENS)
    tab0 = hash[:, 0].astype(jnp.uint32)
    tab1 = hash[:, 1].astype(jnp.uint32)
    r0, r1 = _hash_gather(idx, tab0, tab1)
    h0 = r0.astype(hash.dtype).reshape(BATCH, SEQ)
    h1 = r1.astype(hash.dtype).reshape(BATCH, SEQ)
    return jnp.stack([h0, h1], axis=-1)
